# baseline (speedup 1.0000x reference)
import functools

import jax
import jax.numpy as jnp
from jax import lax
from jax.experimental import pallas as pl
from jax.experimental.pallas import tpu as pltpu
from jax.experimental.pallas import tpu_sc as plsc

ALPHA = 0.01
NC, NS, L = 2, 16, 16
W = 128
TW = 16
CH = 40
TCH = 80


def _leaky(v):
    return jnp.where(v >= 0, v, ALPHA * v)


def _pq_body(x_ref, wp_ref, wq_ref, wpt_ref, wqt_ref,
             p_ref, q_ref, pt_ref, qt_ref):
    xb = x_ref[...]
    p_ref[...] = jnp.dot(xb, wp_ref[0], preferred_element_type=jnp.float32)
    q_ref[...] = jnp.dot(xb, wq_ref[0], preferred_element_type=jnp.float32)
    pt_ref[...] = jnp.dot(xb, wpt_ref[...], preferred_element_type=jnp.float32)
    qt_ref[...] = jnp.dot(xb, wqt_ref[...], preferred_element_type=jnp.float32)


def _redge_main_body(ea_ref, we_ref, bm_ref, r_ref):
    r_ref[...] = (jnp.dot(ea_ref[...], we_ref[0],
                          preferred_element_type=jnp.float32)
                  + bm_ref[0])


def _redge_tail_body(ea8_ref, wek_ref, bmt_ref, rt_ref):
    rt_ref[...] = (jnp.dot(ea8_ref[...], wek_ref[...],
                           preferred_element_type=jnp.float32)
                   + bmt_ref[...])


def _final_body(m0_ref, m1_ref, t0_ref, t1_ref, x_ref,
                w1_ref, w2_ref, w4_ref, w3_ref, bh_ref, h_ref):
    acc = jnp.dot(m0_ref[0], w1_ref[...], preferred_element_type=jnp.float32)
    acc = acc + jnp.dot(m1_ref[0], w2_ref[...], preferred_element_type=jnp.float32)
    mt = t0_ref[0] + t1_ref[0]
    acc = acc + jnp.dot(mt, w4_ref[...], preferred_element_type=jnp.float32)
    acc = acc + jnp.dot(x_ref[...], w3_ref[...], preferred_element_type=jnp.float32)
    acc = acc + bh_ref[...]
    h_ref[...] = _leaky(acc)


def _sc_pipe_body(n, e, npad, w, ch, split_edges, e_off, acc_in_flag,
                  *refs):
    if acc_in_flag:
        (src_h, dst_h, p_h, q_h, r_h, accin_h, out_h,
         srcb0, srcb1, dstb0, dstb1, sgb0, sgb1, dgb0, dgb1,
         dsb0, dsb1, bp0, bp1, bq0, bq1, br0, br1, bm0, bm1,
         acc, sem_idx, sem_gat, sem_sc0, sem_sc1) = refs
    else:
        (src_h, dst_h, p_h, q_h, r_h, out_h,
         srcb0, srcb1, dstb0, dstb1, sgb0, sgb1, dgb0, dgb1,
         dsb0, dsb1, bp0, bp1, bq0, bq1, br0, br1, bm0, bm1,
         acc, sem_idx, sem_gat, sem_sc0, sem_sc1) = refs
        accin_h = None
    srcb = (srcb0, srcb1)
    dstb = (dstb0, dstb1)
    sgb = (sgb0, sgb1)
    dgb = (dgb0, dgb1)
    dsb = (dsb0, dsb1)
    bufp = (bp0, bp1)
    bufq = (bq0, bq1)
    bufr = (br0, br1)
    bufm = (bm0, bm1)
    sem_sc = (sem_sc0, sem_sc1)

    c = lax.axis_index("c")
    s = lax.axis_index("s")
    rows_per_tile = npad // NS
    if split_edges:
        et = e // (NC * NS)
        ebase0 = (c * NS + s) * et
        coff = 0
        rb = 0
    else:
        et = e // NS
        ebase0 = s * et
        coff = c * n
        rb = c * e
    nchunk = et // ch
    npair = nchunk // 2
    odd = nchunk % 2 == 1
    offs = list(range(0, ch - L + 1, L))
    if ch % L:
        offs.append(ch - L)

    r0 = s * rows_per_tile
    if acc_in_flag:
        gbase = c * npad + r0
        for k in range(rows_per_tile // ch):
            pltpu.sync_copy(accin_h.at[pl.ds(gbase + k * ch, ch)],
                            acc.at[pl.ds(r0 + k * ch, ch)])
    else:
        def zrow(i, _):
            for j in range(w // L):
                bufm[0][i, pl.ds(j * L, L)] = jnp.zeros((L,), jnp.float32)
            return 0
        lax.fori_loop(0, ch, zrow, 0)
        for k in range(rows_per_tile // ch):
            pltpu.sync_copy(bufm[0], acc.at[pl.ds(r0 + k * ch, ch)])
    plsc.subcore_barrier()

    def idx_issue(b, base):
        pltpu.async_copy(src_h.at[pl.ds(e_off + base, ch)], srcb[b], sem_idx)
        pltpu.async_copy(dst_h.at[pl.ds(e_off + base, ch)], dstb[b], sem_idx)

    def idx_wait(b):
        pltpu.make_async_copy(src_h.at[pl.ds(0, ch)], srcb[b], sem_idx).wait()
        pltpu.make_async_copy(dst_h.at[pl.ds(0, ch)], dstb[b], sem_idx).wait()

    def r_slice(base):
        if split_edges:
            return r_h.at[pl.ds((rb + base) // 8, ch // 8)]
        return r_h.at[pl.ds(rb + base, ch)]

    def adj_and_gather(b, base):
        for o in offs:
            sl = pl.ds(o, L)
            sgb[b][sl] = srcb[b][sl] + coff
            dgb[b][sl] = dstb[b][sl] + coff
        pltpu.async_copy(p_h.at[sgb[b]], bufp[b], sem_gat)
        pltpu.async_copy(q_h.at[dgb[b]], bufq[b], sem_gat)
        pltpu.async_copy(r_slice(base), bufr[b], sem_gat)

    def gat_wait(b):
        pltpu.make_async_copy(p_h.at[sgb[b]], bufp[b], sem_gat).wait()
        pltpu.make_async_copy(q_h.at[dgb[b]], bufq[b], sem_gat).wait()
        pltpu.make_async_copy(r_slice(ebase0), bufr[b], sem_gat).wait()

    def sc_issue(b):
        pltpu.async_copy(bufm[b], acc.at[dsb[b]], sem_sc[b], add=True)

    def sc_wait(b):
        pltpu.make_async_copy(out_h.at[pl.ds(0, ch)], bufm[b], sem_sc[b]).wait()

    def compute(b):
        for o in offs:
            sl = pl.ds(o, L)
            dsb[b][sl] = dstb[b][sl]

        if split_edges:
            @plsc.parallel_loop(0, ch, unroll=2)
            def _(i):
                sl = pl.ds(0, L)
                v = (bufp[b][i, sl] + bufq[b][i, sl]
                     + bufr[b][i // 8, pl.ds((i % 8) * L, L)])
                bufm[b][i, sl] = jnp.where(v >= 0, v, ALPHA * v)
        else:
            @plsc.parallel_loop(0, ch, unroll=2)
            def _(i):
                for j in range(w // L):
                    sl = pl.ds(j * L, L)
                    v = bufp[b][i, sl] + bufq[b][i, sl] + bufr[b][i, sl]
                    bufm[b][i, sl] = jnp.where(v >= 0, v, ALPHA * v)

    pltpu.sync_copy(src_h.at[pl.ds(e_off + ebase0, ch)], srcb[0])
    pltpu.sync_copy(dst_h.at[pl.ds(e_off + ebase0, ch)], dstb[0])
    adj_and_gather(0, ebase0)
    idx_issue(1, ebase0 + ch)

    def pair(kp, _):
        for b in (0, 1):
            base = ebase0 + (2 * kp + b) * ch
            gat_wait(b)

            @pl.when(kp >= 1)
            def _():
                sc_wait(b)

            if b == 0:
                idx_wait(1)
                adj_and_gather(1, base + ch)
            else:
                if odd:
                    idx_wait(0)
                    adj_and_gather(0, base + ch)
                else:
                    @pl.when(kp <= npair - 2)
                    def _():
                        idx_wait(0)
                        adj_and_gather(0, base + ch)

            compute(b)
            sc_issue(b)

            if (b == 0 and odd):
                idx_issue(b, base + 2 * ch)
            else:
                @pl.when(kp <= npair - 2)
                def _():
                    idx_issue(b, base + 2 * ch)
        return 0

    lax.fori_loop(0, npair, pair, 0)

    if odd:
        gat_wait(0)
        sc_wait(0)
        compute(0)
        sc_issue(0)
        sc_wait(1)
        sc_wait(0)
    else:
        sc_wait(0)
        sc_wait(1)
    plsc.subcore_barrier()

    obase = c * npad + r0
    for k in range(rows_per_tile // ch):
        pltpu.sync_copy(acc.at[pl.ds(r0 + k * ch, ch)], bufm[0])
        pltpu.sync_copy(bufm[0], out_h.at[pl.ds(obase + k * ch, ch)])


def _make_edge_call(n, e, npad, w, ch, split_edges, untiled,
                    e_off=0, acc_in=False):
    mesh = plsc.VectorSubcoreMesh(core_axis_name="c", subcore_axis_name="s")
    i32, f32 = jnp.int32, jnp.float32
    idxbufs = [pltpu.VMEM((ch,), i32) for _ in range(10)]
    rshape = (ch // 8, 8 * w) if split_edges else (ch, w)
    databufs = ([pltpu.VMEM((ch, w), f32) for _ in range(4)]
                + [pltpu.VMEM(rshape, f32) for _ in range(2)]
                + [pltpu.VMEM((ch, w), f32) for _ in range(2)])
    params = pltpu.CompilerParams(use_tc_tiling_on_sc=False) if untiled else None
    return pl.kernel(
        functools.partial(_sc_pipe_body, n, e, npad, w, ch, split_edges,
                          e_off, acc_in),
        out_type=jax.ShapeDtypeStruct((NC * npad, w), f32),
        mesh=mesh,
        scratch_types=idxbufs + databufs + [
            pltpu.VMEM_SHARED((npad, w), f32),
            pltpu.SemaphoreType.DMA,
            pltpu.SemaphoreType.DMA,
            pltpu.SemaphoreType.DMA,
            pltpu.SemaphoreType.DMA,
        ],
        compiler_params=params,
    )


def kernel(x, edge_index, edge_attr, Wm, bm, Wh, bh):
    n, d = x.shape
    e, de = edge_attr.shape
    msg = Wm.shape[0]
    hid = Wh.shape[0]
    npad = ((n + NS * CH * 2 - 1) // (NS * CH * 2)) * (NS * CH * 2)

    f32 = jnp.float32
    wp_full = jnp.transpose(Wm[:, :d])
    wq_full = jnp.transpose(Wm[:, d:2 * d])
    we_full = jnp.transpose(Wm[:, 2 * d:])
    wp_s = jnp.stack([wp_full[:, :W], wp_full[:, W:2 * W]])
    wq_s = jnp.stack([wq_full[:, :W], wq_full[:, W:2 * W]])
    we_s = jnp.stack([we_full[:, :W], we_full[:, W:2 * W]])
    wp_t = wp_full[:, 2 * W:]
    wq_t = wq_full[:, 2 * W:]
    we_t = we_full[:, 2 * W:]
    wek = jnp.kron(jnp.eye(8, dtype=f32), we_t)
    bm_s = jnp.stack([bm[:W], bm[W:2 * W]]).reshape(NC, 1, W)
    bmt8 = jnp.tile(bm[2 * W:], 8).reshape(1, 8 * TW)

    w1t = jnp.transpose(Wh[:, :W])
    w2t = jnp.transpose(Wh[:, W:2 * W])
    w4t = jnp.transpose(Wh[:, 2 * W:msg])
    w3t = jnp.transpose(Wh[:, msg:])
    bh2 = bh.reshape(1, hid)

    src = edge_index[0]
    dst = edge_index[1]
    ea8 = edge_attr.reshape(e // 8, 8 * de)

    p2, q2, pt, qt = pl.pallas_call(
        _pq_body,
        grid=(NC,),
        in_specs=[
            pl.BlockSpec((n, d), lambda c: (0, 0)),
            pl.BlockSpec((1, d, W), lambda c: (c, 0, 0)),
            pl.BlockSpec((1, d, W), lambda c: (c, 0, 0)),
            pl.BlockSpec((d, TW), lambda c: (0, 0)),
            pl.BlockSpec((d, TW), lambda c: (0, 0)),
        ],
        out_specs=[
            pl.BlockSpec((n, W), lambda c: (c, 0)),
            pl.BlockSpec((n, W), lambda c: (c, 0)),
            pl.BlockSpec((n, TW), lambda c: (0, 0)),
            pl.BlockSpec((n, TW), lambda c: (0, 0)),
        ],
        out_shape=[
            jax.ShapeDtypeStruct((NC * n, W), f32),
            jax.ShapeDtypeStruct((NC * n, W), f32),
            jax.ShapeDtypeStruct((n, TW), f32),
            jax.ShapeDtypeStruct((n, TW), f32),
        ],
    )(x, wp_s, wq_s, wp_t, wq_t)

    eb = 3200
    rt8 = pl.pallas_call(
        _redge_tail_body,
        grid=(e // eb,),
        in_specs=[
            pl.BlockSpec((eb // 8, 8 * de), lambda i: (i, 0)),
            pl.BlockSpec((8 * de, 8 * TW), lambda i: (0, 0)),
            pl.BlockSpec((1, 8 * TW), lambda i: (0, 0)),
        ],
        out_specs=pl.BlockSpec((eb // 8, 8 * TW), lambda i: (i, 0)),
        out_shape=jax.ShapeDtypeStruct((e // 8, 8 * TW), f32),
    )(ea8, wek, bmt8)

    def _tail_sum(rt8_d, pt_d, qt_d):
        ts_ = _make_edge_call(n, e, npad, TW, TCH, True, True)(
            src, dst, pt_d, qt_d, rt8_d)
        return ts_.reshape(NC, npad, TW)

    e2 = e // 2
    nbk = e2 // eb

    def _r_half(h):
        return pl.pallas_call(
            _redge_main_body,
            grid=(NC, nbk),
            in_specs=[
                pl.BlockSpec((eb, de), lambda c, i: (h * nbk + i, 0)),
                pl.BlockSpec((1, de, W), lambda c, i: (c, 0, 0)),
                pl.BlockSpec((1, 1, W), lambda c, i: (c, 0, 0)),
            ],
            out_specs=pl.BlockSpec((eb, W), lambda c, i: (c * nbk + i, 0)),
            out_shape=jax.ShapeDtypeStruct((NC * e2, W), f32),
        )(edge_attr, we_s, bm_s)

    r2a = _r_half(0)
    r2b = _r_half(1)

    zacc = jnp.zeros((NC * npad, W), f32)
    msum_a = _make_edge_call(n, e2, npad, W, CH, False, False,
                             e_off=0, acc_in=True)(
        src, dst, p2, q2, r2a, zacc)
    msum = _make_edge_call(n, e2, npad, W, CH, False, False,
                           e_off=e2, acc_in=True)(
        src, dst, p2, q2, r2b, msum_a)
    tsum = _tail_sum(rt8, pt, qt)
    msum = msum.reshape(NC, npad, W)

    nb = 1000
    h = pl.pallas_call(
        _final_body,
        grid=(n // nb,),
        in_specs=[
            pl.BlockSpec((1, nb, W), lambda b: (0, b, 0)),
            pl.BlockSpec((1, nb, W), lambda b: (1, b, 0)),
            pl.BlockSpec((1, nb, TW), lambda b: (0, b, 0)),
            pl.BlockSpec((1, nb, TW), lambda b: (1, b, 0)),
            pl.BlockSpec((nb, d), lambda b: (b, 0)),
            pl.BlockSpec((W, hid), lambda b: (0, 0)),
            pl.BlockSpec((W, hid), lambda b: (0, 0)),
            pl.BlockSpec((TW, hid), lambda b: (0, 0)),
            pl.BlockSpec((d, hid), lambda b: (0, 0)),
            pl.BlockSpec((1, hid), lambda b: (0, 0)),
        ],
        out_specs=pl.BlockSpec((nb, hid), lambda b: (b, 0)),
        out_shape=jax.ShapeDtypeStruct((n, hid), f32),
    )(msum, msum, tsum, tsum, x, w1t, w2t, w4t, w3t, bh2)
    return h


# speedup vs baseline: 1.0143x; 1.0143x over previous
import functools

import jax
import jax.numpy as jnp
from jax import lax
from jax.experimental import pallas as pl
from jax.experimental.pallas import tpu as pltpu
from jax.experimental.pallas import tpu_sc as plsc

ALPHA = 0.01
NC, NS, L = 2, 16, 16
W = 128
TW = 16
CH = 40
TCH = 80


def _leaky(v):
    return jnp.where(v >= 0, v, ALPHA * v)


def _pq_body(x_ref, wp_ref, wq_ref, wpt_ref, wqt_ref,
             p_ref, q_ref, pt_ref, qt_ref):
    xb = x_ref[...]
    p_ref[...] = jnp.dot(xb, wp_ref[0], preferred_element_type=jnp.float32)
    q_ref[...] = jnp.dot(xb, wq_ref[0], preferred_element_type=jnp.float32)
    pt_ref[...] = jnp.dot(xb, wpt_ref[...], preferred_element_type=jnp.float32)
    qt_ref[...] = jnp.dot(xb, wqt_ref[...], preferred_element_type=jnp.float32)


def _redge_main_body(ea_ref, we_ref, bm_ref, r_ref):
    r_ref[...] = (jnp.dot(ea_ref[...], we_ref[0],
                          preferred_element_type=jnp.float32)
                  + bm_ref[0])


def _redge_tail_body(ea8_ref, wek_ref, bmt_ref, rt_ref):
    rt_ref[...] = (jnp.dot(ea8_ref[...], wek_ref[...],
                           preferred_element_type=jnp.float32)
                   + bmt_ref[...])


def _final_body(m0_ref, m1_ref, t0_ref, t1_ref, x_ref,
                w1_ref, w2_ref, w4_ref, w3_ref, bh_ref, h_ref):
    acc = jnp.dot(m0_ref[0], w1_ref[...], preferred_element_type=jnp.float32)
    acc = acc + jnp.dot(m1_ref[0], w2_ref[...], preferred_element_type=jnp.float32)
    mt = t0_ref[0] + t1_ref[0]
    acc = acc + jnp.dot(mt, w4_ref[...], preferred_element_type=jnp.float32)
    acc = acc + jnp.dot(x_ref[...], w3_ref[...], preferred_element_type=jnp.float32)
    acc = acc + bh_ref[...]
    h_ref[...] = _leaky(acc)


def _sc_pipe_body(n, e, npad, w, ch, split_edges, e_off, acc_in_flag,
                  *refs):
    if acc_in_flag:
        (src_h, dst_h, p_h, q_h, r_h, accin_h, out_h,
         srcb0, srcb1, dstb0, dstb1, sgb0, sgb1, dgb0, dgb1,
         dsb0, dsb1, bp0, bp1, bq0, bq1, br0, br1, bm0, bm1,
         acc, sem_idx, sem_gat, sem_sc0, sem_sc1) = refs
    else:
        (src_h, dst_h, p_h, q_h, r_h, out_h,
         srcb0, srcb1, dstb0, dstb1, sgb0, sgb1, dgb0, dgb1,
         dsb0, dsb1, bp0, bp1, bq0, bq1, br0, br1, bm0, bm1,
         acc, sem_idx, sem_gat, sem_sc0, sem_sc1) = refs
        accin_h = None
    srcb = (srcb0, srcb1)
    dstb = (dstb0, dstb1)
    sgb = (sgb0, sgb1)
    dgb = (dgb0, dgb1)
    dsb = (dsb0, dsb1)
    bufp = (bp0, bp1)
    bufq = (bq0, bq1)
    bufr = (br0, br1)
    bufm = (bm0, bm1)
    sem_sc = (sem_sc0, sem_sc1)

    c = lax.axis_index("c")
    s = lax.axis_index("s")
    rows_per_tile = npad // NS
    if split_edges:
        et = e // (NC * NS)
        ebase0 = (c * NS + s) * et
        coff = 0
        rb = 0
    else:
        et = e // NS
        ebase0 = s * et
        coff = c * n
        rb = c * e
    nchunk = et // ch
    npair = nchunk // 2
    odd = nchunk % 2 == 1
    offs = list(range(0, ch - L + 1, L))
    if ch % L:
        offs.append(ch - L)

    r0 = s * rows_per_tile
    if acc_in_flag:
        gbase = c * npad + r0
        for k in range(rows_per_tile // ch):
            pltpu.sync_copy(accin_h.at[pl.ds(gbase + k * ch, ch)],
                            acc.at[pl.ds(r0 + k * ch, ch)])
    else:
        def zrow(i, _):
            for j in range(w // L):
                bufm[0][i, pl.ds(j * L, L)] = jnp.zeros((L,), jnp.float32)
            return 0
        lax.fori_loop(0, ch, zrow, 0)
        for k in range(rows_per_tile // ch):
            pltpu.sync_copy(bufm[0], acc.at[pl.ds(r0 + k * ch, ch)])
    plsc.subcore_barrier()

    def idx_issue(b, base):
        pltpu.async_copy(src_h.at[pl.ds(e_off + base, ch)], srcb[b], sem_idx)
        pltpu.async_copy(dst_h.at[pl.ds(e_off + base, ch)], dstb[b], sem_idx)

    def idx_wait(b):
        pltpu.make_async_copy(src_h.at[pl.ds(0, ch)], srcb[b], sem_idx).wait()
        pltpu.make_async_copy(dst_h.at[pl.ds(0, ch)], dstb[b], sem_idx).wait()

    def r_slice(base):
        if split_edges:
            return r_h.at[pl.ds((rb + base) // 8, ch // 8)]
        return r_h.at[pl.ds(rb + base, ch)]

    def adj_and_gather(b, base):
        for o in offs:
            sl = pl.ds(o, L)
            sgb[b][sl] = srcb[b][sl] + coff
            dgb[b][sl] = dstb[b][sl] + coff
        pltpu.async_copy(p_h.at[sgb[b]], bufp[b], sem_gat)
        pltpu.async_copy(q_h.at[dgb[b]], bufq[b], sem_gat)
        pltpu.async_copy(r_slice(base), bufr[b], sem_gat)

    def gat_wait(b):
        pltpu.make_async_copy(p_h.at[sgb[b]], bufp[b], sem_gat).wait()
        pltpu.make_async_copy(q_h.at[dgb[b]], bufq[b], sem_gat).wait()
        pltpu.make_async_copy(r_slice(ebase0), bufr[b], sem_gat).wait()

    def sc_issue(b):
        pltpu.async_copy(bufm[b], acc.at[dsb[b]], sem_sc[b], add=True)

    def sc_wait(b):
        pltpu.make_async_copy(out_h.at[pl.ds(0, ch)], bufm[b], sem_sc[b]).wait()

    def compute(b):
        for o in offs:
            sl = pl.ds(o, L)
            dsb[b][sl] = dstb[b][sl]

        if split_edges:
            @plsc.parallel_loop(0, ch, unroll=2)
            def _(i):
                sl = pl.ds(0, L)
                v = (bufp[b][i, sl] + bufq[b][i, sl]
                     + bufr[b][i // 8, pl.ds((i % 8) * L, L)])
                bufm[b][i, sl] = jnp.where(v >= 0, v, ALPHA * v)
        else:
            @plsc.parallel_loop(0, ch, unroll=2)
            def _(i):
                for j in range(w // L):
                    sl = pl.ds(j * L, L)
                    v = bufp[b][i, sl] + bufq[b][i, sl] + bufr[b][i, sl]
                    bufm[b][i, sl] = jnp.where(v >= 0, v, ALPHA * v)

    pltpu.sync_copy(src_h.at[pl.ds(e_off + ebase0, ch)], srcb[0])
    pltpu.sync_copy(dst_h.at[pl.ds(e_off + ebase0, ch)], dstb[0])
    adj_and_gather(0, ebase0)
    idx_issue(1, ebase0 + ch)

    def pair(kp, _):
        for b in (0, 1):
            base = ebase0 + (2 * kp + b) * ch
            gat_wait(b)

            @pl.when(kp >= 1)
            def _():
                sc_wait(b)

            if b == 0:
                idx_wait(1)
                adj_and_gather(1, base + ch)
            else:
                if odd:
                    idx_wait(0)
                    adj_and_gather(0, base + ch)
                else:
                    @pl.when(kp <= npair - 2)
                    def _():
                        idx_wait(0)
                        adj_and_gather(0, base + ch)

            compute(b)
            sc_issue(b)

            if (b == 0 and odd):
                idx_issue(b, base + 2 * ch)
            else:
                @pl.when(kp <= npair - 2)
                def _():
                    idx_issue(b, base + 2 * ch)
        return 0

    lax.fori_loop(0, npair, pair, 0)

    if odd:
        gat_wait(0)
        sc_wait(0)
        compute(0)
        sc_issue(0)
        sc_wait(1)
        sc_wait(0)
    else:
        sc_wait(0)
        sc_wait(1)
    plsc.subcore_barrier()

    obase = c * npad + r0
    for k in range(rows_per_tile // ch):
        pltpu.sync_copy(acc.at[pl.ds(r0 + k * ch, ch)], bufm[0])
        pltpu.sync_copy(bufm[0], out_h.at[pl.ds(obase + k * ch, ch)])


def _make_edge_call(n, e, npad, w, ch, split_edges, untiled,
                    e_off=0, acc_in=False):
    mesh = plsc.VectorSubcoreMesh(core_axis_name="c", subcore_axis_name="s")
    i32, f32 = jnp.int32, jnp.float32
    idxbufs = [pltpu.VMEM((ch,), i32) for _ in range(10)]
    rshape = (ch // 8, 8 * w) if split_edges else (ch, w)
    databufs = ([pltpu.VMEM((ch, w), f32) for _ in range(4)]
                + [pltpu.VMEM(rshape, f32) for _ in range(2)]
                + [pltpu.VMEM((ch, w), f32) for _ in range(2)])
    params = pltpu.CompilerParams(use_tc_tiling_on_sc=False) if untiled else None
    return pl.kernel(
        functools.partial(_sc_pipe_body, n, e, npad, w, ch, split_edges,
                          e_off, acc_in),
        out_type=jax.ShapeDtypeStruct((NC * npad, w), f32),
        mesh=mesh,
        scratch_types=idxbufs + databufs + [
            pltpu.VMEM_SHARED((npad, w), f32),
            pltpu.SemaphoreType.DMA,
            pltpu.SemaphoreType.DMA,
            pltpu.SemaphoreType.DMA,
            pltpu.SemaphoreType.DMA,
        ],
        compiler_params=params,
    )


def kernel(x, edge_index, edge_attr, Wm, bm, Wh, bh):
    n, d = x.shape
    e, de = edge_attr.shape
    msg = Wm.shape[0]
    hid = Wh.shape[0]
    npad = ((n + NS * CH * 2 - 1) // (NS * CH * 2)) * (NS * CH * 2)

    f32 = jnp.float32
    wp_full = jnp.transpose(Wm[:, :d])
    wq_full = jnp.transpose(Wm[:, d:2 * d])
    we_full = jnp.transpose(Wm[:, 2 * d:])
    wp_s = jnp.stack([wp_full[:, :W], wp_full[:, W:2 * W]])
    wq_s = jnp.stack([wq_full[:, :W], wq_full[:, W:2 * W]])
    we_s = jnp.stack([we_full[:, :W], we_full[:, W:2 * W]])
    wp_t = wp_full[:, 2 * W:]
    wq_t = wq_full[:, 2 * W:]
    we_t = we_full[:, 2 * W:]
    wek = jnp.kron(jnp.eye(8, dtype=f32), we_t)
    bm_s = jnp.stack([bm[:W], bm[W:2 * W]]).reshape(NC, 1, W)
    bmt8 = jnp.tile(bm[2 * W:], 8).reshape(1, 8 * TW)

    w1t = jnp.transpose(Wh[:, :W])
    w2t = jnp.transpose(Wh[:, W:2 * W])
    w4t = jnp.transpose(Wh[:, 2 * W:msg])
    w3t = jnp.transpose(Wh[:, msg:])
    bh2 = bh.reshape(1, hid)

    src = edge_index[0]
    dst = edge_index[1]
    ea8 = edge_attr.reshape(e // 8, 8 * de)

    p2, q2, pt, qt = pl.pallas_call(
        _pq_body,
        grid=(NC,),
        in_specs=[
            pl.BlockSpec((n, d), lambda c: (0, 0)),
            pl.BlockSpec((1, d, W), lambda c: (c, 0, 0)),
            pl.BlockSpec((1, d, W), lambda c: (c, 0, 0)),
            pl.BlockSpec((d, TW), lambda c: (0, 0)),
            pl.BlockSpec((d, TW), lambda c: (0, 0)),
        ],
        out_specs=[
            pl.BlockSpec((n, W), lambda c: (c, 0)),
            pl.BlockSpec((n, W), lambda c: (c, 0)),
            pl.BlockSpec((n, TW), lambda c: (0, 0)),
            pl.BlockSpec((n, TW), lambda c: (0, 0)),
        ],
        out_shape=[
            jax.ShapeDtypeStruct((NC * n, W), f32),
            jax.ShapeDtypeStruct((NC * n, W), f32),
            jax.ShapeDtypeStruct((n, TW), f32),
            jax.ShapeDtypeStruct((n, TW), f32),
        ],
    )(x, wp_s, wq_s, wp_t, wq_t)

    eb = 3200
    rt8 = pl.pallas_call(
        _redge_tail_body,
        grid=(e // eb,),
        in_specs=[
            pl.BlockSpec((eb // 8, 8 * de), lambda i: (i, 0)),
            pl.BlockSpec((8 * de, 8 * TW), lambda i: (0, 0)),
            pl.BlockSpec((1, 8 * TW), lambda i: (0, 0)),
        ],
        out_specs=pl.BlockSpec((eb // 8, 8 * TW), lambda i: (i, 0)),
        out_shape=jax.ShapeDtypeStruct((e // 8, 8 * TW), f32),
    )(ea8, wek, bmt8)

    def _tail_sum(rt8_d, pt_d, qt_d):
        ts_ = _make_edge_call(n, e, npad, TW, TCH, True, True)(
            src, dst, pt_d, qt_d, rt8_d)
        return ts_.reshape(NC, npad, TW)

    e2 = e // 2
    nbk = e2 // eb

    def _r_half(h):
        return pl.pallas_call(
            _redge_main_body,
            grid=(NC, nbk),
            in_specs=[
                pl.BlockSpec((eb, de), lambda c, i: (h * nbk + i, 0)),
                pl.BlockSpec((1, de, W), lambda c, i: (c, 0, 0)),
                pl.BlockSpec((1, 1, W), lambda c, i: (c, 0, 0)),
            ],
            out_specs=pl.BlockSpec((eb, W), lambda c, i: (c * nbk + i, 0)),
            out_shape=jax.ShapeDtypeStruct((NC * e2, W), f32),
        )(edge_attr, we_s, bm_s)

    r2a = _r_half(0)
    r2b = _r_half(1)

    msum_a = _make_edge_call(n, e2, npad, W, CH, False, False)(
        src, dst, p2, q2, r2a)
    msum = _make_edge_call(n, e2, npad, W, CH, False, False,
                           e_off=e2, acc_in=True)(
        src, dst, p2, q2, r2b, msum_a)
    tsum = _tail_sum(rt8, pt, qt)
    msum = msum.reshape(NC, npad, W)

    nb = 1000
    h = pl.pallas_call(
        _final_body,
        grid=(n // nb,),
        in_specs=[
            pl.BlockSpec((1, nb, W), lambda b: (0, b, 0)),
            pl.BlockSpec((1, nb, W), lambda b: (1, b, 0)),
            pl.BlockSpec((1, nb, TW), lambda b: (0, b, 0)),
            pl.BlockSpec((1, nb, TW), lambda b: (1, b, 0)),
            pl.BlockSpec((nb, d), lambda b: (b, 0)),
            pl.BlockSpec((W, hid), lambda b: (0, 0)),
            pl.BlockSpec((W, hid), lambda b: (0, 0)),
            pl.BlockSpec((TW, hid), lambda b: (0, 0)),
            pl.BlockSpec((d, hid), lambda b: (0, 0)),
            pl.BlockSpec((1, hid), lambda b: (0, 0)),
        ],
        out_specs=pl.BlockSpec((nb, hid), lambda b: (b, 0)),
        out_shape=jax.ShapeDtypeStruct((n, hid), f32),
    )(msum, msum, tsum, tsum, x, w1t, w2t, w4t, w3t, bh2)
    return h
